# baseline (speedup 1.0000x reference)
import jax
import jax.numpy as jnp
from jax import lax
from jax.experimental import pallas as pl
from jax.experimental.pallas import tpu as pltpu
from jax.experimental.pallas import tpu_sc as plsc

N = 10000
E = 320000
H = 128

NC = 2
NS = 16
NW = NC * NS
C = 80
T = 128
EP = NW * T * C
NBO = N // 400


def _sc_agg_kernel(x_hbm, src_hbm, dst_hbm, out0, out1,
                   sv0, sv1, sv2, dv0, dv1, dv2, rows0, rows1, acc_sh,
                   sem_g, sem_s0, sem_s1, sem_i0, sem_i1, sem_i2):
    c = lax.axis_index("c")
    s = lax.axis_index("s")
    wid = c * NS + s
    sv = (sv0, sv1, sv2)
    dv = (dv0, dv1, dv2)
    rows = (rows0, rows1)
    sem_s = (sem_s0, sem_s1)
    sem_i = (sem_i0, sem_i1, sem_i2)

    def zbody(r, _):
        def zcol(j, _):
            rows0[r, pl.ds(j * 16, 16)] = jnp.zeros((16,), jnp.float32)
            return 0

        lax.fori_loop(0, H // 16, zcol, 0)
        return 0

    lax.fori_loop(0, C, zbody, 0)
    NZB = N // C
    for j in range((NZB + NS - 1) // NS):
        blk = s + j * NS

        @pl.when(blk < NZB)
        def _():
            off = pl.multiple_of(blk * C, 8)
            pltpu.sync_copy(rows0, acc_sh.at[pl.ds(off, C)])

    @pl.when(s == NS - 1)
    def _():
        rem = N - NZB * C
        pltpu.sync_copy(rows0.at[pl.ds(0, rem)],
                        acc_sh.at[pl.ds(NZB * C, rem)])

    plsc.subcore_barrier()

    cb = wid * T

    def body(t, _):
        off = pl.multiple_of((cb + t) * C, 8)
        pltpu.sync_copy(src_hbm.at[pl.ds(off, C)], sv0)
        pltpu.sync_copy(dst_hbm.at[pl.ds(off, C)], dv0)
        pltpu.async_copy(x_hbm.at[sv0], rows0, sem_g).wait()
        pltpu.sync_copy(rows0, acc_sh.at[dv0], add=True)
        return 0

    lax.fori_loop(0, T, body, 0)
    plsc.subcore_barrier()

    for j in range(2):
        blk = s + j * NS

        @pl.when(blk < NBO)
        def _():
            off = pl.multiple_of(blk * 400, 8)
            sl = pl.ds(off, 400)

            @pl.when(c == 0)
            def _():
                pltpu.sync_copy(acc_sh.at[sl], out0.at[sl])

            @pl.when(c == 1)
            def _():
                pltpu.sync_copy(acc_sh.at[sl], out1.at[sl])


def _sc_agg(x, src1, dst1):
    mesh = plsc.VectorSubcoreMesh(core_axis_name="c", subcore_axis_name="s",
                                  num_cores=NC, num_subcores=NS)
    f = pl.kernel(
        _sc_agg_kernel,
        out_type=(jax.ShapeDtypeStruct((N, H), jnp.float32),
                  jax.ShapeDtypeStruct((N, H), jnp.float32)),
        mesh=mesh,
        scratch_types=[
            pltpu.VMEM((C,), jnp.int32),
            pltpu.VMEM((C,), jnp.int32),
            pltpu.VMEM((C,), jnp.int32),
            pltpu.VMEM((C,), jnp.int32),
            pltpu.VMEM((C,), jnp.int32),
            pltpu.VMEM((C,), jnp.int32),
            pltpu.VMEM((C, H), jnp.float32),
            pltpu.VMEM((C, H), jnp.float32),
            pltpu.VMEM_SHARED((N + 8, H), jnp.float32),
            pltpu.SemaphoreType.DMA,
            pltpu.SemaphoreType.DMA,
            pltpu.SemaphoreType.DMA,
            pltpu.SemaphoreType.DMA,
            pltpu.SemaphoreType.DMA,
            pltpu.SemaphoreType.DMA,
        ],
    )
    return f(x, src1, dst1)


def _pad_idx(idx, fill):
    return jnp.concatenate([idx, jnp.full((EP - E,), fill, jnp.int32)])


def _mlp_body(eps_ref, x_ref, a0_ref, a1_ref, W1_ref, g_ref, b_ref, W2_ref,
              lng_ref, lnb_ref, dir_ref, o_ref):
    x = x_ref[...]
    h = x * (1.0 + eps_ref[0]) + a0_ref[...] + a1_ref[...]
    h = jnp.dot(h, W1_ref[...], preferred_element_type=jnp.float32)
    m = jnp.mean(h, axis=-1, keepdims=True)
    v = jnp.mean((h - m) * (h - m), axis=-1, keepdims=True)
    h = (h - m) * lax.rsqrt(v + 1e-5) * g_ref[...] + b_ref[...]
    h = jnp.maximum(h, 0.0)
    h = jnp.dot(h, W2_ref[...], preferred_element_type=jnp.float32)
    y = jnp.maximum(h + x + dir_ref[...], 0.0)
    m2 = jnp.mean(y, axis=-1, keepdims=True)
    v2 = jnp.mean((y - m2) * (y - m2), axis=-1, keepdims=True)
    o_ref[...] = (y - m2) * lax.rsqrt(v2 + 1e-5) * lng_ref[...] + lnb_ref[...]


BN = 1000


def _mlp(x, a0, a1, eps, W1, g, b, W2, lng, lnb, dir_row):
    grid = (N // BN,)
    row_spec = pl.BlockSpec((BN, H), lambda i: (i, 0))
    full = lambda a: pl.BlockSpec(a.shape, lambda i: (0,) * a.ndim)
    g_, b_ = g.reshape(1, -1), b.reshape(1, -1)
    lng_, lnb_ = lng.reshape(1, -1), lnb.reshape(1, -1)
    dir_ = dir_row.reshape(1, -1)
    return pl.pallas_call(
        _mlp_body,
        grid=grid,
        in_specs=[
            pl.BlockSpec(memory_space=pltpu.SMEM),
            row_spec, row_spec, row_spec,
            full(W1), full(g_), full(b_), full(W2),
            full(lng_), full(lnb_), full(dir_),
        ],
        out_specs=row_spec,
        out_shape=jax.ShapeDtypeStruct((N, H), jnp.float32),
    )(eps.reshape(1), x, a0, a1, W1, g_, b_, W2, lng_, lnb_, dir_)


def kernel(x, edge_index, eps_d, W1_d, g_d, b_d, W2_d, eps_u, W1_u, g_u,
           b_u, W2_u, ln1_g, ln1_b, ln2_g, ln2_b, dir_emb):
    src = edge_index[0].astype(jnp.int32)
    dst = edge_index[1].astype(jnp.int32)
    src_g = _pad_idx(src, 0)
    dst_s = _pad_idx(dst, N)
    dst_g = _pad_idx(dst, 0)
    src_s = _pad_idx(src, N)
    a0, a1 = _sc_agg(x, src_g, dst_s)
    x1 = _mlp(x, a0, a1, eps_d, W1_d, g_d, b_d, W2_d, ln1_g, ln1_b, dir_emb[0])
    b0, b1 = _sc_agg(x1, dst_g, src_s)
    x2 = _mlp(x1, b0, b1, eps_u, W1_u, g_u, b_u, W2_u, ln2_g, ln2_b, dir_emb[1])
    return x2


# speedup vs baseline: 1.9231x; 1.9231x over previous
import functools

import jax
import jax.numpy as jnp
from jax import lax
from jax.experimental import pallas as pl
from jax.experimental.pallas import tpu as pltpu
from jax.experimental.pallas import tpu_sc as plsc

N = 10000
E = 320000
H = 128

NC = 2
NS = 16
NW = NC * NS
EPW = E // NW
C = 80
ITERS = EPW // C
BLK = 200
NB = N // BLK


def _sc_agg_kernel(x_hbm, src_hbm, dst_hbm, out0, out1,
                   src_v, dst_v, rows_v, zero_v, acc_sh, sem):
    c = lax.axis_index("c")
    s = lax.axis_index("s")
    wid = c * NS + s

    def zbody(r, _):
        def zcol(j, _):
            zero_v[r, pl.ds(j * 16, 16)] = jnp.zeros((16,), jnp.float32)
            return 0

        lax.fori_loop(0, H // 16, zcol, 0)
        return 0

    lax.fori_loop(0, BLK, zbody, 0)
    for j in range((NB + NS - 1) // NS):
        k = s + j * NS

        @pl.when(k < NB)
        def _():
            off = pl.multiple_of(k * BLK, 8)
            pltpu.sync_copy(zero_v, acc_sh.at[pl.ds(off, BLK)])

    plsc.subcore_barrier()

    def body(i, _):
        base = pl.multiple_of(wid * EPW + i * C, 8)
        pltpu.sync_copy(src_hbm.at[pl.ds(base, C)], src_v)
        pltpu.sync_copy(dst_hbm.at[pl.ds(base, C)], dst_v)
        pltpu.async_copy(x_hbm.at[src_v], rows_v, sem).wait()
        pltpu.sync_copy(rows_v, acc_sh.at[dst_v], add=True)
        return 0

    lax.fori_loop(0, ITERS, body, 0)
    plsc.subcore_barrier()

    for j in range((NB + NS - 1) // NS):
        k = s + j * NS

        @pl.when(k < NB)
        def _():
            off = pl.multiple_of(k * BLK, 8)
            sl = pl.ds(off, BLK)

            @pl.when(c == 0)
            def _():
                pltpu.sync_copy(acc_sh.at[sl], out0.at[sl])

            @pl.when(c == 1)
            def _():
                pltpu.sync_copy(acc_sh.at[sl], out1.at[sl])


def _sc_agg(x, src, dst):
    mesh = plsc.VectorSubcoreMesh(core_axis_name="c", subcore_axis_name="s",
                                  num_cores=NC, num_subcores=NS)
    f = pl.kernel(
        _sc_agg_kernel,
        out_type=(jax.ShapeDtypeStruct((N, H), jnp.float32),
                  jax.ShapeDtypeStruct((N, H), jnp.float32)),
        mesh=mesh,
        scratch_types=[
            pltpu.VMEM((C,), jnp.int32),
            pltpu.VMEM((C,), jnp.int32),
            pltpu.VMEM((C, H), jnp.float32),
            pltpu.VMEM((BLK, H), jnp.float32),
            pltpu.VMEM_SHARED((N, H), jnp.float32),
            pltpu.SemaphoreType.DMA,
        ],
    )
    return f(x, src, dst)


def _mlp_body(eps_ref, x_ref, a0_ref, a1_ref, W1_ref, g_ref, b_ref, W2_ref,
              lng_ref, lnb_ref, dir_ref, o_ref):
    x = x_ref[...]
    h = x * (1.0 + eps_ref[0]) + a0_ref[...] + a1_ref[...]
    h = jnp.dot(h, W1_ref[...], preferred_element_type=jnp.float32)
    m = jnp.mean(h, axis=-1, keepdims=True)
    v = jnp.mean((h - m) * (h - m), axis=-1, keepdims=True)
    h = (h - m) * lax.rsqrt(v + 1e-5) * g_ref[...] + b_ref[...]
    h = jnp.maximum(h, 0.0)
    h = jnp.dot(h, W2_ref[...], preferred_element_type=jnp.float32)
    y = jnp.maximum(h + x + dir_ref[...], 0.0)
    m2 = jnp.mean(y, axis=-1, keepdims=True)
    v2 = jnp.mean((y - m2) * (y - m2), axis=-1, keepdims=True)
    o_ref[...] = (y - m2) * lax.rsqrt(v2 + 1e-5) * lng_ref[...] + lnb_ref[...]


BN = 1000


def _mlp(x, a0, a1, eps, W1, g, b, W2, lng, lnb, dir_row):
    grid = (N // BN,)
    row_spec = pl.BlockSpec((BN, H), lambda i: (i, 0))
    full = lambda a: pl.BlockSpec(a.shape, lambda i: (0,) * a.ndim)
    g_, b_ = g.reshape(1, -1), b.reshape(1, -1)
    lng_, lnb_ = lng.reshape(1, -1), lnb.reshape(1, -1)
    dir_ = dir_row.reshape(1, -1)
    return pl.pallas_call(
        _mlp_body,
        grid=grid,
        in_specs=[
            pl.BlockSpec(memory_space=pltpu.SMEM),
            row_spec, row_spec, row_spec,
            full(W1), full(g_), full(b_), full(W2),
            full(lng_), full(lnb_), full(dir_),
        ],
        out_specs=row_spec,
        out_shape=jax.ShapeDtypeStruct((N, H), jnp.float32),
    )(eps.reshape(1), x, a0, a1, W1, g_, b_, W2, lng_, lnb_, dir_)


def kernel(x, edge_index, eps_d, W1_d, g_d, b_d, W2_d, eps_u, W1_u, g_u,
           b_u, W2_u, ln1_g, ln1_b, ln2_g, ln2_b, dir_emb):
    src = edge_index[0].astype(jnp.int32)
    dst = edge_index[1].astype(jnp.int32)
    a0, a1 = _sc_agg(x, src, dst)
    x1 = _mlp(x, a0, a1, eps_d, W1_d, g_d, b_d, W2_d, ln1_g, ln1_b, dir_emb[0])
    b0, b1 = _sc_agg(x1, dst, src)
    x2 = _mlp(x1, b0, b1, eps_u, W1_u, g_u, b_u, W2_u, ln2_g, ln2_b, dir_emb[1])
    return x2


# speedup vs baseline: 2.1860x; 1.1367x over previous
import functools

import jax
import jax.numpy as jnp
from jax import lax
from jax.experimental import pallas as pl
from jax.experimental.pallas import tpu as pltpu
from jax.experimental.pallas import tpu_sc as plsc

N = 10000
E = 320000
H = 128

NC = 2
NS = 16
NW = NC * NS
EPW = E // NW
C = 80
ITERS = EPW // C
BLK = 200
NB = N // BLK


def _sc_agg_kernel(x_hbm, idx_hbm, out0, out1,
                   si_v, rows_v, zero_v, acc_sh, sem):
    c = lax.axis_index("c")
    s = lax.axis_index("s")
    wid = c * NS + s

    def zbody(r, _):
        def zcol(j, _):
            zero_v[r, pl.ds(j * 16, 16)] = jnp.zeros((16,), jnp.float32)
            return 0

        lax.fori_loop(0, H // 16, zcol, 0)
        return 0

    lax.fori_loop(0, BLK, zbody, 0)
    for j in range((NB + NS - 1) // NS):
        k = s + j * NS

        @pl.when(k < NB)
        def _():
            off = pl.multiple_of(k * BLK, 8)
            pltpu.sync_copy(zero_v, acc_sh.at[pl.ds(off, BLK)])

    plsc.subcore_barrier()

    def body(i, _):
        base = pl.multiple_of((wid * ITERS + i) * 2 * C, 8)
        pltpu.sync_copy(idx_hbm.at[pl.ds(base, 2 * C)], si_v)
        pltpu.async_copy(x_hbm.at[si_v.at[pl.ds(0, C)]], rows_v, sem).wait()
        pltpu.sync_copy(rows_v, acc_sh.at[si_v.at[pl.ds(C, C)]], add=True)
        return 0

    lax.fori_loop(0, ITERS, body, 0)
    plsc.subcore_barrier()

    for j in range((NB + NS - 1) // NS):
        k = s + j * NS

        @pl.when(k < NB)
        def _():
            off = pl.multiple_of(k * BLK, 8)
            sl = pl.ds(off, BLK)

            @pl.when(c == 0)
            def _():
                pltpu.sync_copy(acc_sh.at[sl], out0.at[sl])

            @pl.when(c == 1)
            def _():
                pltpu.sync_copy(acc_sh.at[sl], out1.at[sl])


def _sc_agg(x, idx):
    mesh = plsc.VectorSubcoreMesh(core_axis_name="c", subcore_axis_name="s",
                                  num_cores=NC, num_subcores=NS)
    f = pl.kernel(
        _sc_agg_kernel,
        out_type=(jax.ShapeDtypeStruct((N, H), jnp.float32),
                  jax.ShapeDtypeStruct((N, H), jnp.float32)),
        mesh=mesh,
        scratch_types=[
            pltpu.VMEM((2 * C,), jnp.int32),
            pltpu.VMEM((C, H), jnp.float32),
            pltpu.VMEM((BLK, H), jnp.float32),
            pltpu.VMEM_SHARED((N, H), jnp.float32),
            pltpu.SemaphoreType.DMA,
        ],
    )
    return f(x, idx)


def _mlp_body(eps_ref, x_ref, a0_ref, a1_ref, W1_ref, g_ref, b_ref, W2_ref,
              lng_ref, lnb_ref, dir_ref, o_ref):
    x = x_ref[...]
    h = x * (1.0 + eps_ref[0]) + a0_ref[...] + a1_ref[...]
    h = jnp.dot(h, W1_ref[...], preferred_element_type=jnp.float32)
    m = jnp.mean(h, axis=-1, keepdims=True)
    v = jnp.mean((h - m) * (h - m), axis=-1, keepdims=True)
    h = (h - m) * lax.rsqrt(v + 1e-5) * g_ref[...] + b_ref[...]
    h = jnp.maximum(h, 0.0)
    h = jnp.dot(h, W2_ref[...], preferred_element_type=jnp.float32)
    y = jnp.maximum(h + x + dir_ref[...], 0.0)
    m2 = jnp.mean(y, axis=-1, keepdims=True)
    v2 = jnp.mean((y - m2) * (y - m2), axis=-1, keepdims=True)
    o_ref[...] = (y - m2) * lax.rsqrt(v2 + 1e-5) * lng_ref[...] + lnb_ref[...]


BN = 1000


def _mlp(x, a0, a1, eps, W1, g, b, W2, lng, lnb, dir_row):
    grid = (N // BN,)
    row_spec = pl.BlockSpec((BN, H), lambda i: (i, 0))
    full = lambda a: pl.BlockSpec(a.shape, lambda i: (0,) * a.ndim)
    g_, b_ = g.reshape(1, -1), b.reshape(1, -1)
    lng_, lnb_ = lng.reshape(1, -1), lnb.reshape(1, -1)
    dir_ = dir_row.reshape(1, -1)
    return pl.pallas_call(
        _mlp_body,
        grid=grid,
        in_specs=[
            pl.BlockSpec(memory_space=pltpu.SMEM),
            row_spec, row_spec, row_spec,
            full(W1), full(g_), full(b_), full(W2),
            full(lng_), full(lnb_), full(dir_),
        ],
        out_specs=row_spec,
        out_shape=jax.ShapeDtypeStruct((N, H), jnp.float32),
    )(eps.reshape(1), x, a0, a1, W1, g_, b_, W2, lng_, lnb_, dir_)


def _pack_idx(gather_idx, scatter_idx):
    g = gather_idx.reshape(E // C, C)
    s = scatter_idx.reshape(E // C, C)
    return jnp.concatenate([g, s], axis=1).reshape(-1)


def kernel(x, edge_index, eps_d, W1_d, g_d, b_d, W2_d, eps_u, W1_u, g_u,
           b_u, W2_u, ln1_g, ln1_b, ln2_g, ln2_b, dir_emb):
    src = edge_index[0].astype(jnp.int32)
    dst = edge_index[1].astype(jnp.int32)
    a0, a1 = _sc_agg(x, _pack_idx(src, dst))
    x1 = _mlp(x, a0, a1, eps_d, W1_d, g_d, b_d, W2_d, ln1_g, ln1_b, dir_emb[0])
    b0, b1 = _sc_agg(x1, _pack_idx(dst, src))
    x2 = _mlp(x1, b0, b1, eps_u, W1_u, g_u, b_u, W2_u, ln2_g, ln2_b, dir_emb[1])
    return x2


# speedup vs baseline: 2.6792x; 1.2256x over previous
import functools

import jax
import jax.numpy as jnp
from jax import lax
from jax.experimental import pallas as pl
from jax.experimental.pallas import tpu as pltpu
from jax.experimental.pallas import tpu_sc as plsc

N = 10000
E = 320000
H = 128

NC = 2
NS = 16
NW = NC * NS
EPW = E // NW
C = 80
ITERS = EPW // C
BLK = 200
NB = N // BLK


def _sc_agg_kernel(x_hbm, idx_hbm, out0, out1,
                   si0, si1, rows_v, zero_v, acc_sh, sem, sem_i0, sem_i1):
    c = lax.axis_index("c")
    s = lax.axis_index("s")
    wid = c * NS + s

    def zbody(r, _):
        def zcol(j, _):
            zero_v[r, pl.ds(j * 16, 16)] = jnp.zeros((16,), jnp.float32)
            return 0

        lax.fori_loop(0, H // 16, zcol, 0)
        return 0

    lax.fori_loop(0, BLK, zbody, 0)
    for j in range((NB + NS - 1) // NS):
        k = s + j * NS

        @pl.when(k < NB)
        def _():
            off = pl.multiple_of(k * BLK, 8)
            pltpu.sync_copy(zero_v, acc_sh.at[pl.ds(off, BLK)])

    plsc.subcore_barrier()

    si = (si0, si1)
    sem_i = (sem_i0, sem_i1)

    def issue_idx(i, S):
        base = pl.multiple_of((wid * ITERS + i) * 2 * C, 8)
        pltpu.async_copy(idx_hbm.at[pl.ds(base, 2 * C)], si[S], sem_i[S])

    def drain_idx(S):
        base0 = pl.multiple_of(wid * ITERS * 2 * C, 8)
        pltpu.make_async_copy(idx_hbm.at[pl.ds(base0, 2 * C)], si[S],
                              sem_i[S]).wait()

    issue_idx(0, 0)
    drain_idx(0)

    def step(i, S):
        So = 1 - S

        @pl.when(i + 1 < ITERS)
        def _():
            issue_idx(i + 1, So)

        pltpu.async_copy(x_hbm.at[si[S].at[pl.ds(0, C)]], rows_v, sem).wait()
        pltpu.sync_copy(rows_v, acc_sh.at[si[S].at[pl.ds(C, C)]], add=True)

        @pl.when(i + 1 < ITERS)
        def _():
            drain_idx(So)

    def body(i, _):
        @pl.when(i % 2 == 0)
        def _():
            step(i, 0)

        @pl.when(i % 2 == 1)
        def _():
            step(i, 1)

        return 0

    lax.fori_loop(0, ITERS, body, 0)
    plsc.subcore_barrier()

    for j in range((NB + NS - 1) // NS):
        k = s + j * NS

        @pl.when(k < NB)
        def _():
            off = pl.multiple_of(k * BLK, 8)
            sl = pl.ds(off, BLK)

            @pl.when(c == 0)
            def _():
                pltpu.sync_copy(acc_sh.at[sl], out0.at[sl])

            @pl.when(c == 1)
            def _():
                pltpu.sync_copy(acc_sh.at[sl], out1.at[sl])


def _sc_agg(x, idx):
    mesh = plsc.VectorSubcoreMesh(core_axis_name="c", subcore_axis_name="s",
                                  num_cores=NC, num_subcores=NS)
    f = pl.kernel(
        _sc_agg_kernel,
        out_type=(jax.ShapeDtypeStruct((N, H), jnp.float32),
                  jax.ShapeDtypeStruct((N, H), jnp.float32)),
        mesh=mesh,
        scratch_types=[
            pltpu.VMEM((2 * C,), jnp.int32),
            pltpu.VMEM((2 * C,), jnp.int32),
            pltpu.VMEM((C, H), jnp.float32),
            pltpu.VMEM((BLK, H), jnp.float32),
            pltpu.VMEM_SHARED((N, H), jnp.float32),
            pltpu.SemaphoreType.DMA,
            pltpu.SemaphoreType.DMA,
            pltpu.SemaphoreType.DMA,
        ],
    )
    return f(x, idx)


def _mlp_body(eps_ref, x_ref, a0_ref, a1_ref, W1_ref, g_ref, b_ref, W2_ref,
              lng_ref, lnb_ref, dir_ref, o_ref):
    x = x_ref[...]
    h = x * (1.0 + eps_ref[0]) + a0_ref[...] + a1_ref[...]
    h = jnp.dot(h, W1_ref[...], preferred_element_type=jnp.float32)
    m = jnp.mean(h, axis=-1, keepdims=True)
    v = jnp.mean((h - m) * (h - m), axis=-1, keepdims=True)
    h = (h - m) * lax.rsqrt(v + 1e-5) * g_ref[...] + b_ref[...]
    h = jnp.maximum(h, 0.0)
    h = jnp.dot(h, W2_ref[...], preferred_element_type=jnp.float32)
    y = jnp.maximum(h + x + dir_ref[...], 0.0)
    m2 = jnp.mean(y, axis=-1, keepdims=True)
    v2 = jnp.mean((y - m2) * (y - m2), axis=-1, keepdims=True)
    o_ref[...] = (y - m2) * lax.rsqrt(v2 + 1e-5) * lng_ref[...] + lnb_ref[...]


BN = 1000


def _mlp(x, a0, a1, eps, W1, g, b, W2, lng, lnb, dir_row):
    grid = (N // BN,)
    row_spec = pl.BlockSpec((BN, H), lambda i: (i, 0))
    full = lambda a: pl.BlockSpec(a.shape, lambda i: (0,) * a.ndim)
    g_, b_ = g.reshape(1, -1), b.reshape(1, -1)
    lng_, lnb_ = lng.reshape(1, -1), lnb.reshape(1, -1)
    dir_ = dir_row.reshape(1, -1)
    return pl.pallas_call(
        _mlp_body,
        grid=grid,
        in_specs=[
            pl.BlockSpec(memory_space=pltpu.SMEM),
            row_spec, row_spec, row_spec,
            full(W1), full(g_), full(b_), full(W2),
            full(lng_), full(lnb_), full(dir_),
        ],
        out_specs=row_spec,
        out_shape=jax.ShapeDtypeStruct((N, H), jnp.float32),
    )(eps.reshape(1), x, a0, a1, W1, g_, b_, W2, lng_, lnb_, dir_)


def _pack_idx(gather_idx, scatter_idx):
    g = gather_idx.reshape(E // C, C)
    s = scatter_idx.reshape(E // C, C)
    return jnp.concatenate([g, s], axis=1).reshape(-1)


def kernel(x, edge_index, eps_d, W1_d, g_d, b_d, W2_d, eps_u, W1_u, g_u,
           b_u, W2_u, ln1_g, ln1_b, ln2_g, ln2_b, dir_emb):
    src = edge_index[0].astype(jnp.int32)
    dst = edge_index[1].astype(jnp.int32)
    a0, a1 = _sc_agg(x, _pack_idx(src, dst))
    x1 = _mlp(x, a0, a1, eps_d, W1_d, g_d, b_d, W2_d, ln1_g, ln1_b, dir_emb[0])
    b0, b1 = _sc_agg(x1, _pack_idx(dst, src))
    x2 = _mlp(x1, b0, b1, eps_u, W1_u, g_u, b_u, W2_u, ln2_g, ln2_b, dir_emb[1])
    return x2


# speedup vs baseline: 3.3618x; 1.2548x over previous
import functools

import jax
import jax.numpy as jnp
from jax import lax
from jax.experimental import pallas as pl
from jax.experimental.pallas import tpu as pltpu
from jax.experimental.pallas import tpu_sc as plsc

N = 10000
E = 320000
H = 128

NC = 2
NS = 16
NW = NC * NS
EPW = E // NW
C = 80
ITERS = EPW // C
BLK = 200
NB = N // BLK


def _sc_agg_kernel(x_hbm, idx_hbm, out0, out1, si0, si1, sd0, sd1,
                   rows0, rows1, zero_v, acc_sh, sem, sem_i0, sem_i1,
                   sem_s0, sem_s1):
    c = lax.axis_index("c")
    s = lax.axis_index("s")
    wid = c * NS + s

    def zbody(r, _):
        def zcol(j, _):
            zero_v[r, pl.ds(j * 16, 16)] = jnp.zeros((16,), jnp.float32)
            return 0

        lax.fori_loop(0, H // 16, zcol, 0)
        return 0

    lax.fori_loop(0, BLK, zbody, 0)
    for j in range((NB + NS - 1) // NS):
        k = s + j * NS

        @pl.when(k < NB)
        def _():
            off = pl.multiple_of(k * BLK, 8)
            pltpu.sync_copy(zero_v, acc_sh.at[pl.ds(off, BLK)])

    plsc.subcore_barrier()

    si = (si0, si1)
    sd = (sd0, sd1)
    rows = (rows0, rows1)
    sem_i = (sem_i0, sem_i1)
    sem_s = (sem_s0, sem_s1)

    def issue_idx(i, S):
        base = pl.multiple_of((wid * ITERS + i) * 2 * C, 8)
        pltpu.async_copy(idx_hbm.at[pl.ds(base, 2 * C)], si[S], sem_i[S])

    def drain_idx(S):
        base0 = pl.multiple_of(wid * ITERS * 2 * C, 8)
        pltpu.make_async_copy(idx_hbm.at[pl.ds(base0, 2 * C)], si[S],
                              sem_i[S]).wait()

    issue_idx(0, 0)
    drain_idx(0)

    def step(i, S):
        So = 1 - S

        @pl.when(i + 1 < ITERS)
        def _():
            issue_idx(i + 1, So)

        pltpu.async_copy(x_hbm.at[si[S].at[pl.ds(0, C)]], rows[S], sem).wait()
        for j in range(C // 16):
            sd[S][pl.ds(j * 16, 16)] = si[S][pl.ds(C + j * 16, 16)]
        pltpu.async_copy(rows[S], acc_sh.at[sd[S]], sem_s[S], add=True)

        @pl.when(i >= 1)
        def _():
            pltpu.make_async_copy(rows[So], acc_sh.at[sd[So]],
                                  sem_s[So]).wait()

        @pl.when(i + 1 < ITERS)
        def _():
            drain_idx(So)

    def body(i, _):
        @pl.when(i % 2 == 0)
        def _():
            step(i, 0)

        @pl.when(i % 2 == 1)
        def _():
            step(i, 1)

        return 0

    lax.fori_loop(0, ITERS, body, 0)
    Sl = (ITERS - 1) % 2
    pltpu.make_async_copy(rows[Sl], acc_sh.at[sd[Sl]], sem_s[Sl]).wait()
    plsc.subcore_barrier()

    for j in range((NB + NS - 1) // NS):
        k = s + j * NS

        @pl.when(k < NB)
        def _():
            off = pl.multiple_of(k * BLK, 8)
            sl = pl.ds(off, BLK)

            @pl.when(c == 0)
            def _():
                pltpu.sync_copy(acc_sh.at[sl], out0.at[sl])

            @pl.when(c == 1)
            def _():
                pltpu.sync_copy(acc_sh.at[sl], out1.at[sl])


def _sc_agg(x, idx):
    mesh = plsc.VectorSubcoreMesh(core_axis_name="c", subcore_axis_name="s",
                                  num_cores=NC, num_subcores=NS)
    f = pl.kernel(
        _sc_agg_kernel,
        out_type=(jax.ShapeDtypeStruct((N, H), jnp.float32),
                  jax.ShapeDtypeStruct((N, H), jnp.float32)),
        mesh=mesh,
        scratch_types=[
            pltpu.VMEM((2 * C,), jnp.int32),
            pltpu.VMEM((2 * C,), jnp.int32),
            pltpu.VMEM((C,), jnp.int32),
            pltpu.VMEM((C,), jnp.int32),
            pltpu.VMEM((C, H), jnp.float32),
            pltpu.VMEM((C, H), jnp.float32),
            pltpu.VMEM((BLK, H), jnp.float32),
            pltpu.VMEM_SHARED((N, H), jnp.float32),
            pltpu.SemaphoreType.DMA,
            pltpu.SemaphoreType.DMA,
            pltpu.SemaphoreType.DMA,
            pltpu.SemaphoreType.DMA,
            pltpu.SemaphoreType.DMA,
        ],
    )
    return f(x, idx)


def _mlp_body(eps_ref, x_ref, a0_ref, a1_ref, W1_ref, g_ref, b_ref, W2_ref,
              lng_ref, lnb_ref, dir_ref, o_ref):
    x = x_ref[...]
    h = x * (1.0 + eps_ref[0]) + a0_ref[...] + a1_ref[...]
    h = jnp.dot(h, W1_ref[...], preferred_element_type=jnp.float32)
    m = jnp.mean(h, axis=-1, keepdims=True)
    v = jnp.mean((h - m) * (h - m), axis=-1, keepdims=True)
    h = (h - m) * lax.rsqrt(v + 1e-5) * g_ref[...] + b_ref[...]
    h = jnp.maximum(h, 0.0)
    h = jnp.dot(h, W2_ref[...], preferred_element_type=jnp.float32)
    y = jnp.maximum(h + x + dir_ref[...], 0.0)
    m2 = jnp.mean(y, axis=-1, keepdims=True)
    v2 = jnp.mean((y - m2) * (y - m2), axis=-1, keepdims=True)
    o_ref[...] = (y - m2) * lax.rsqrt(v2 + 1e-5) * lng_ref[...] + lnb_ref[...]


BN = 1000


def _mlp(x, a0, a1, eps, W1, g, b, W2, lng, lnb, dir_row):
    grid = (N // BN,)
    row_spec = pl.BlockSpec((BN, H), lambda i: (i, 0))
    full = lambda a: pl.BlockSpec(a.shape, lambda i: (0,) * a.ndim)
    g_, b_ = g.reshape(1, -1), b.reshape(1, -1)
    lng_, lnb_ = lng.reshape(1, -1), lnb.reshape(1, -1)
    dir_ = dir_row.reshape(1, -1)
    return pl.pallas_call(
        _mlp_body,
        grid=grid,
        in_specs=[
            pl.BlockSpec(memory_space=pltpu.SMEM),
            row_spec, row_spec, row_spec,
            full(W1), full(g_), full(b_), full(W2),
            full(lng_), full(lnb_), full(dir_),
        ],
        out_specs=row_spec,
        out_shape=jax.ShapeDtypeStruct((N, H), jnp.float32),
    )(eps.reshape(1), x, a0, a1, W1, g_, b_, W2, lng_, lnb_, dir_)


def _pack_idx(gather_idx, scatter_idx):
    g = gather_idx.reshape(E // C, C)
    s = scatter_idx.reshape(E // C, C)
    return jnp.concatenate([g, s], axis=1).reshape(-1)


def kernel(x, edge_index, eps_d, W1_d, g_d, b_d, W2_d, eps_u, W1_u, g_u,
           b_u, W2_u, ln1_g, ln1_b, ln2_g, ln2_b, dir_emb):
    src = edge_index[0].astype(jnp.int32)
    dst = edge_index[1].astype(jnp.int32)
    a0, a1 = _sc_agg(x, _pack_idx(src, dst))
    x1 = _mlp(x, a0, a1, eps_d, W1_d, g_d, b_d, W2_d, ln1_g, ln1_b, dir_emb[0])
    b0, b1 = _sc_agg(x1, _pack_idx(dst, src))
    x2 = _mlp(x1, b0, b1, eps_u, W1_u, g_u, b_u, W2_u, ln2_g, ln2_b, dir_emb[1])
    return x2
